# baseline (speedup 1.0000x reference)
import functools

import jax
import jax.numpy as jnp
from jax import lax
from jax.experimental import pallas as pl
from jax.experimental.pallas import tpu as pltpu
from jax.experimental.pallas import tpu_sc as plsc

SEP = 7
LANES = 16


def _remain_sc_body(seq, nwork, di_hbm, w2s_hbm, out_hbm, di_v, w2s_v, rem_v):
    c = lax.axis_index("c")
    s = lax.axis_index("s")
    wid = s * 2 + c

    def full(v):
        return jnp.full((LANES,), v, jnp.int32)

    @pl.when(wid < nwork)
    def _():
        col0 = wid * LANES
        pltpu.sync_copy(di_hbm, di_v)
        pltpu.sync_copy(w2s_hbm, w2s_v)
        c0, c2, c5, c7 = full(0), full(2), full(5), full(7)
        rs = c0
        seg = c0
        for t in range(seq):
            tok = di_v[t, pl.ds(col0, LANES)]
            is_sep = tok == c7
            sep_i = jnp.where(is_sep, full(1), c0)
            if t == 0:
                seg = sep_i
                rs = jnp.where(is_sep, c7, c5)
            else:
                syl = plsc.load_gather(w2s_v, [tok])
                rs = jnp.maximum(rs - syl, c0)
                seg = jnp.minimum(seg + sep_i, c5)
                pat = jnp.where(seg == c5, c0,
                                jnp.where((seg == c0) | (seg == c2), c5, c7))
                rs = jnp.where(is_sep, pat, rs)
            rem_v[t] = rs
        pltpu.sync_copy(rem_v, out_hbm.at[wid])


def _mask_body(rem_ref, w2s_ref, logits_ref, out_ref):
    out_ref[0] = jnp.where(w2s_ref[:] > rem_ref[0], -jnp.inf, logits_ref[0])


def kernel(logits, decoder_input, word2syllables):
    seq, batch = decoder_input.shape
    vocab = logits.shape[-1]
    nwork = batch // LANES

    vpad = (-vocab) % LANES
    w2s_pad = jnp.concatenate(
        [word2syllables, jnp.zeros((vpad,), word2syllables.dtype)])

    remain_fn = pl.kernel(
        functools.partial(_remain_sc_body, seq, nwork),
        out_type=jax.ShapeDtypeStruct((nwork, seq, LANES), jnp.int32),
        mesh=plsc.VectorSubcoreMesh(core_axis_name="c", subcore_axis_name="s"),
        compiler_params=pltpu.CompilerParams(needs_layout_passes=False),
        scratch_types=[
            pltpu.VMEM((seq, batch), jnp.int32),
            pltpu.VMEM((vocab + vpad,), jnp.int32),
            pltpu.VMEM((seq, LANES), jnp.int32),
        ],
    )
    remain_w = remain_fn(decoder_input, w2s_pad)
    remain = jnp.transpose(remain_w, (1, 0, 2)).reshape(seq, batch)

    remain3 = remain.reshape(seq, batch, 1)
    w2s2 = word2syllables.reshape(1, vocab)
    out = pl.pallas_call(
        _mask_body,
        grid=(seq,),
        in_specs=[
            pl.BlockSpec((1, batch, 1), lambda i: (i, 0, 0)),
            pl.BlockSpec((1, vocab), lambda i: (0, 0)),
            pl.BlockSpec((1, batch, vocab), lambda i: (i, 0, 0)),
        ],
        out_specs=pl.BlockSpec((1, batch, vocab), lambda i: (i, 0, 0)),
        out_shape=jax.ShapeDtypeStruct((seq, batch, vocab), jnp.float32),
    )(remain3, w2s2, logits)
    return out
